# baseline (speedup 1.0000x reference)
import jax
import jax.numpy as jnp
from jax.experimental import pallas as pl
from jax.experimental.pallas import tpu as pltpu

CHANNELS = 768
TOPK = 384
TEMP = 1.0
C_BLK = 128
N_CBLK = CHANNELS // C_BLK
NB = 16
H = 56
W = 56
D = 6
DZ = 4
TOTAL = N_CBLK * NB


def _mask_kernel(logits_ref, m_ref, meta_ref):
    lg = logits_ref[0, :]
    col = lg[None, :]
    row = lg[:, None]
    i_idx = jax.lax.broadcasted_iota(jnp.int32, (CHANNELS, CHANNELS), 0)
    j_idx = jax.lax.broadcasted_iota(jnp.int32, (CHANNELS, CHANNELS), 1)
    beats = (col > row) | ((col == row) & (j_idx < i_idx))
    rank = jnp.sum(beats.astype(jnp.int32), axis=1)
    hard = (rank < TOPK).astype(jnp.float32)
    soft = jax.nn.sigmoid(lg / TEMP)
    m = (hard - soft) + soft
    m_ref[0, :] = m

    act = (jnp.sum(hard.reshape(N_CBLK, C_BLK), axis=1) > 0).astype(jnp.int32)
    a_col = act[None, :]
    ci = jax.lax.broadcasted_iota(jnp.int32, (N_CBLK, N_CBLK), 0)
    cj = jax.lax.broadcasted_iota(jnp.int32, (N_CBLK, N_CBLK), 1)
    inc = jnp.sum(jnp.where(cj <= ci, a_col, 0), axis=1)
    num_active = jnp.sum(act)
    c_lin = jax.lax.broadcasted_iota(jnp.int32, (1, N_CBLK), 1)[0]
    pos = jnp.where(act == 1, inc - 1, num_active + c_lin - inc)
    perm = jnp.sum(jnp.where(pos[None, :] == ci, cj, 0), axis=1)

    c_sub = jax.lax.broadcasted_iota(jnp.int32, (N_CBLK, 128), 0)
    lane2 = jax.lax.broadcasted_iota(jnp.int32, (N_CBLK, 128), 1)
    meta = jnp.sum(jnp.where(lane2 == c_sub + 8, perm[:, None], 0), axis=0)
    lane = jax.lax.broadcasted_iota(jnp.int32, (1, 128), 1)
    meta = meta + jnp.where(lane[0] == 7, num_active, 0)
    meta_ref[0, :] = meta


def _gate_kernel(meta_ref, m_ref, z_ref, out_ref,
                 zbuf, obuf, zerobuf, in_sems, out_sems, z_sems):
    num_active = meta_ref[0, 7]
    active_t = num_active * NB

    def chan_of(q):
        return meta_ref[0, 8 + q // NB] * C_BLK

    def in_copy(q, slot):
        return pltpu.make_async_copy(
            z_ref.at[pl.ds(q % NB, 1), pl.ds(chan_of(q), C_BLK), :, :],
            zbuf.at[pl.ds(slot, 1)],
            in_sems.at[slot])

    def out_copy(q, slot):
        return pltpu.make_async_copy(
            obuf.at[pl.ds(slot, 1)],
            out_ref.at[pl.ds(q % NB, 1), pl.ds(chan_of(q), C_BLK), :, :],
            out_sems.at[slot])

    def zero_copy(q, slot):
        return pltpu.make_async_copy(
            zerobuf,
            out_ref.at[pl.ds(q % NB, 1), pl.ds(chan_of(q), C_BLK), :, :],
            z_sems.at[slot])

    zerobuf[...] = jnp.zeros((1, C_BLK, H, W), jnp.float32)

    for slot in range(D):
        in_copy(slot, slot).start()

    def active_body(q, _):
        slot = q % D
        in_copy(q, slot).wait()

        @pl.when(q >= D)
        def _():
            out_copy(q - D, slot).wait()

        pc = meta_ref[0, 8 + q // NB]
        obuf[pl.ds(slot, 1)] = zbuf[pl.ds(slot, 1)] * m_ref[pc][None, :, None, None]
        out_copy(q, slot).start()

        @pl.when(q + D < active_t)
        def _():
            in_copy(q + D, slot).start()
        return 0

    jax.lax.fori_loop(0, active_t, active_body, 0)

    def zero_body(q, _):
        slot = q % DZ

        @pl.when(q - active_t >= DZ)
        def _():
            zero_copy(q - DZ, slot).wait()

        zero_copy(q, slot).start()
        return 0

    jax.lax.fori_loop(active_t, TOTAL, zero_body, 0)

    def drain_out(i, _):
        q = active_t - D + i
        out_copy(q, q % D).wait()
        return 0

    jax.lax.fori_loop(0, D, drain_out, 0)

    def drain_zero(i, _):
        q = TOTAL - i - 1

        @pl.when(q >= active_t)
        def _():
            zero_copy(q, q % DZ).wait()
        return 0

    jax.lax.fori_loop(0, DZ, drain_zero, 0)


def kernel(z, logits):
    m_out, meta = pl.pallas_call(
        _mask_kernel,
        out_shape=(
            jax.ShapeDtypeStruct((1, CHANNELS), jnp.float32),
            jax.ShapeDtypeStruct((1, 128), jnp.int32),
        ),
    )(logits.reshape(1, CHANNELS))
    m2 = m_out.reshape(N_CBLK, C_BLK)

    out = pl.pallas_call(
        _gate_kernel,
        in_specs=[
            pl.BlockSpec(memory_space=pltpu.MemorySpace.SMEM),
            pl.BlockSpec(memory_space=pltpu.MemorySpace.VMEM),
            pl.BlockSpec(memory_space=pl.ANY),
        ],
        out_specs=pl.BlockSpec(memory_space=pl.ANY),
        out_shape=jax.ShapeDtypeStruct((NB, CHANNELS, H, W), jnp.float32),
        scratch_shapes=[
            pltpu.VMEM((D, C_BLK, H, W), jnp.float32),
            pltpu.VMEM((D, C_BLK, H, W), jnp.float32),
            pltpu.VMEM((1, C_BLK, H, W), jnp.float32),
            pltpu.SemaphoreType.DMA((D,)),
            pltpu.SemaphoreType.DMA((D,)),
            pltpu.SemaphoreType.DMA((DZ,)),
        ],
    )(meta, m2, z)
    return out


# speedup vs baseline: 6.2333x; 6.2333x over previous
import jax
import jax.numpy as jnp
from jax.experimental import pallas as pl
from jax.experimental.pallas import tpu as pltpu

CHANNELS = 768
TOPK = 384
TEMP = 1.0
NB = 16
H = 56
W = 56
ROWS = NB * H * W
R_BLK = 1024
N_RBLK = ROWS // R_BLK


def _mask_kernel(logits_ref, m_ref):
    lg = logits_ref[0, :]
    col = lg[None, :]
    row = lg[:, None]
    i_idx = jax.lax.broadcasted_iota(jnp.int32, (CHANNELS, CHANNELS), 0)
    j_idx = jax.lax.broadcasted_iota(jnp.int32, (CHANNELS, CHANNELS), 1)
    beats = (col > row) | ((col == row) & (j_idx < i_idx))
    rank = jnp.sum(beats.astype(jnp.int32), axis=1)
    hard = (rank < TOPK).astype(jnp.float32)
    soft = jax.nn.sigmoid(lg / TEMP)
    m = (hard - soft) + soft
    m_ref[0, :] = m


def _gate_kernel(z_ref, m_ref, out_ref):
    out_ref[...] = z_ref[...] * m_ref[0][None, :]


def kernel(z, logits):
    zt = z.transpose(0, 2, 3, 1).reshape(ROWS, CHANNELS)
    m_out = pl.pallas_call(
        _mask_kernel,
        out_shape=jax.ShapeDtypeStruct((1, CHANNELS), jnp.float32),
    )(logits.reshape(1, CHANNELS))

    out = pl.pallas_call(
        _gate_kernel,
        grid=(N_RBLK,),
        in_specs=[
            pl.BlockSpec((R_BLK, CHANNELS), lambda i: (i, 0)),
            pl.BlockSpec((1, CHANNELS), lambda i: (0, 0)),
        ],
        out_specs=pl.BlockSpec((R_BLK, CHANNELS), lambda i: (i, 0)),
        out_shape=jax.ShapeDtypeStruct((ROWS, CHANNELS), jnp.float32),
    )(zt, m_out)
    return out.reshape(NB, H, W, CHANNELS).transpose(0, 3, 1, 2)
